# baseline (speedup 1.0000x reference)
import jax
import jax.numpy as jnp
from jax import lax
from jax.experimental import pallas as pl
from jax.experimental.pallas import tpu as pltpu
from jax.experimental.pallas import tpu_sc as plsc

H = 16
P = 2048
NREL = 2 * P - 1
SRC_LEN = 3968
VLEN = 4088
NG = 16
K_PER_PAIR = 16
NBUF = 4
SPLIT = 1


def _make_sc_body(p_base, pairs):
    def _sc_body(src_hbm, out_hbm, buf, so0, so1, so2, so3, si0, si1, si2, si3):
        c = lax.axis_index("c")
        h = lax.axis_index("s")
        so = (so0, so1, so2, so3)
        si = (si0, si1, si2, si3)

        def in_copy(p):
            b = p % NBUF
            return pltpu.make_async_copy(
                src_hbm.at[h, c * pairs + p], buf.at[b], si[b]
            )

        def drain_outs(b):
            for _ in range(K_PER_PAIR):
                pltpu.make_async_copy(
                    buf.at[b, :, pl.ds(0, P)],
                    out_hbm.at[0, h, pl.ds(0, 8), :],
                    so[b],
                ).wait()

        for q in range(min(NBUF - 1, pairs)):
            in_copy(q).start()
        for p in range(pairs):
            b = p % NBUF
            in_copy(p).wait()
            for k in range(K_PER_PAIR):
                g = 8 * c + p_base + p
                i0 = pl.multiple_of(2040 - 8 * g - 128 * k, 8)
                pltpu.make_async_copy(
                    buf.at[b, :, pl.ds(128 * k, P)],
                    out_hbm.at[0, h, pl.ds(i0, 8), :],
                    so[b],
                ).start()
            nxt = p + NBUF - 1
            if nxt < pairs:
                if p >= 1:
                    drain_outs(nxt % NBUF)
                in_copy(nxt).start()

        for pane in range(max(0, pairs - NBUF), pairs):
            drain_outs(pane % NBUF)

    return _sc_body


_SCRATCH = [pltpu.VMEM((NBUF, 8, SRC_LEN), jnp.float32)] + [
    pltpu.SemaphoreType.DMA
] * (2 * NBUF)


@jax.jit
def _rpb(table):
    rev = jnp.flip(jnp.pad(table, ((1, 0), (0, 0))), axis=0).T
    vflip = jnp.stack([rev[:, 7 - r:7 - r + VLEN] for r in range(8)], axis=1)

    def panes(p_base, pairs):
        gs = [8 * c + p_base + p for c in range(2) for p in range(pairs)]
        return jnp.stack(
            [vflip[:, :, 8 * g:8 * g + SRC_LEN] for g in gs], axis=1
        )

    mesh = plsc.VectorSubcoreMesh(core_axis_name="c", subcore_axis_name="s")
    out_a = pl.kernel(
        _make_sc_body(0, SPLIT),
        out_type=jax.ShapeDtypeStruct((1, H, P, P), jnp.float32),
        mesh=mesh,
        scratch_types=_SCRATCH,
    )(panes(0, SPLIT))

    out_ref = jax.new_ref(out_a)
    pl.kernel(
        _make_sc_body(SPLIT, 8 - SPLIT),
        out_type=(),
        mesh=mesh,
        scratch_types=_SCRATCH,
    )(panes(SPLIT, 8 - SPLIT), out_ref)
    return out_ref[...]


def kernel(q_len, k_len, table):
    return _rpb(table)


# speedup vs baseline: 1.1344x; 1.1344x over previous
import jax
import jax.numpy as jnp
from jax import lax
from jax.experimental import pallas as pl
from jax.experimental.pallas import tpu as pltpu
from jax.experimental.pallas import tpu_sc as plsc

H = 16
P = 2048
NREL = 2 * P - 1
SRC_LEN = 3968
VLEN = 4088
NG = 16
K_PER_PAIR = 16
NBUF = 4
SPLIT = 2


def _make_sc_body(p_base, pairs):
    def _sc_body(src_hbm, out_hbm, buf, so0, so1, so2, so3, si0, si1, si2, si3):
        c = lax.axis_index("c")
        h = lax.axis_index("s")
        so = (so0, so1, so2, so3)
        si = (si0, si1, si2, si3)

        def in_copy(p):
            b = p % NBUF
            return pltpu.make_async_copy(
                src_hbm.at[h, c * pairs + p], buf.at[b], si[b]
            )

        def drain_outs(b):
            for _ in range(K_PER_PAIR):
                pltpu.make_async_copy(
                    buf.at[b, :, pl.ds(0, P)],
                    out_hbm.at[0, h, pl.ds(0, 8), :],
                    so[b],
                ).wait()

        for q in range(min(NBUF - 1, pairs)):
            in_copy(q).start()
        for p in range(pairs):
            b = p % NBUF
            in_copy(p).wait()
            for k in range(K_PER_PAIR):
                g = 8 * c + p_base + p
                i0 = pl.multiple_of(2040 - 8 * g - 128 * k, 8)
                pltpu.make_async_copy(
                    buf.at[b, :, pl.ds(128 * k, P)],
                    out_hbm.at[0, h, pl.ds(i0, 8), :],
                    so[b],
                ).start()
            nxt = p + NBUF - 1
            if nxt < pairs:
                if p >= 1:
                    drain_outs(nxt % NBUF)
                in_copy(nxt).start()

        for pane in range(max(0, pairs - NBUF), pairs):
            drain_outs(pane % NBUF)

    return _sc_body


_SCRATCH = [pltpu.VMEM((NBUF, 8, SRC_LEN), jnp.float32)] + [
    pltpu.SemaphoreType.DMA
] * (2 * NBUF)


@jax.jit
def _rpb(table):
    rev = jnp.flip(jnp.pad(table, ((1, 0), (0, 0))), axis=0).T
    vflip = jnp.stack([rev[:, 7 - r:7 - r + VLEN] for r in range(8)], axis=1)

    def panes(p_base, pairs):
        gs = [8 * c + p_base + p for c in range(2) for p in range(pairs)]
        return jnp.stack(
            [vflip[:, :, 8 * g:8 * g + SRC_LEN] for g in gs], axis=1
        )

    mesh = plsc.VectorSubcoreMesh(core_axis_name="c", subcore_axis_name="s")
    out_a = pl.kernel(
        _make_sc_body(0, SPLIT),
        out_type=jax.ShapeDtypeStruct((1, H, P, P), jnp.float32),
        mesh=mesh,
        scratch_types=_SCRATCH,
    )(panes(0, SPLIT))

    out_ref = jax.new_ref(out_a)
    pl.kernel(
        _make_sc_body(SPLIT, 8 - SPLIT),
        out_type=(),
        mesh=mesh,
        scratch_types=_SCRATCH,
    )(panes(SPLIT, 8 - SPLIT), out_ref)
    return out_ref[...]


def kernel(q_len, k_len, table):
    return _rpb(table)
